# baseline (speedup 1.0000x reference)
import functools

import jax
import jax.numpy as jnp
from jax import lax
from jax.experimental import pallas as pl
from jax.experimental.pallas import tpu as pltpu
from jax.experimental.pallas import tpu_sc as plsc

_NC = 2
_NS = 16


def _build_sc_gather(B, D, name):
    nw = _NC * _NS
    b_per_w = B // nw
    half = b_per_w // 2
    assert B % (8 * nw) == 0 and D % 16 == 0

    mesh = plsc.VectorSubcoreMesh(core_axis_name="c", subcore_axis_name="s")

    @functools.partial(
        pl.kernel,
        mesh=mesh,
        name=name,
        compiler_params=pltpu.CompilerParams(
            use_tc_tiling_on_sc=False,
            skip_device_barrier=True,
        ),
        out_type=jax.ShapeDtypeStruct((B, D), jnp.float32),
        scratch_types=[
            pltpu.VMEM((b_per_w,), jnp.int32),
            pltpu.VMEM((half, D), jnp.float32),
            pltpu.VMEM((half, D), jnp.float32),
            pltpu.SemaphoreType.DMA,
            pltpu.SemaphoreType.DMA,
        ],
    )
    def _gather(idx_hbm, tab_hbm, out, idx_v, rows_a, rows_b, sem_a, sem_b):
        wid = lax.axis_index("s") * _NC + lax.axis_index("c")
        base = wid * b_per_w
        pltpu.sync_copy(idx_hbm.at[pl.ds(base, b_per_w)], idx_v)
        ca = pltpu.async_copy(tab_hbm.at[idx_v.at[pl.ds(0, half)]],
                              rows_a, sem_a)
        cb = pltpu.async_copy(tab_hbm.at[idx_v.at[pl.ds(half, half)]],
                              rows_b, sem_b)
        ca.wait()
        pltpu.sync_copy(rows_a, out.at[pl.ds(base, half)])
        cb.wait()
        pltpu.sync_copy(rows_b, out.at[pl.ds(base + half, half)])

    return _gather


def kernel(user_id, item_id, user_emb, item_emb):
    B = user_id.shape[0]
    D = user_emb.shape[1]
    g_u = _build_sc_gather(B, D, "user_gather")
    g_i = _build_sc_gather(B, D, "item_gather")
    u = g_u(user_id.astype(jnp.int32), user_emb)
    i = g_i(item_id.astype(jnp.int32), item_emb)
    return (u, i)
